# Initial kernel scaffold; baseline (speedup 1.0000x reference)
import jax
import jax.numpy as jnp
from jax.experimental import pallas as pl


def kernel(edge_attr, W0, W1, W2):
    raise NotImplementedError("write your pallas kernel here")


# speedup vs baseline: 3.5085x; 3.5085x over previous
import functools

import jax
import jax.numpy as jnp
from jax import lax
from jax.experimental import pallas as pl
from jax.experimental.pallas import tpu as pltpu
from jax.experimental.pallas import tpu_sc as plsc

D0, D1, D2 = 5, 6, 2
NCOMB = D0 * D1 * D2
EMB = 32
N_EDGES = 1600000
L = 16

B = 400


def _make_kernel(num_cores, num_subcores):
  nw = num_cores * num_subcores
  per_w = N_EDGES // nw
  n_chunks = per_w // B
  assert per_w * nw == N_EDGES and n_chunks * B == per_w and B % L == 0

  mesh = plsc.VectorSubcoreMesh(core_axis_name="c", subcore_axis_name="s")

  @functools.partial(
      pl.kernel,
      out_type=jax.ShapeDtypeStruct((N_EDGES, EMB), jnp.float32),
      mesh=mesh,
      compiler_params=pltpu.CompilerParams(needs_layout_passes=False),
      scratch_types=[
          pltpu.VMEM((D0, EMB), jnp.float32),
          pltpu.VMEM((D1, EMB), jnp.float32),
          pltpu.VMEM((D2, EMB), jnp.float32),
          pltpu.VMEM((NCOMB, EMB), jnp.float32),
          pltpu.VMEM((B, 3), jnp.int32),
          pltpu.VMEM((B, EMB), jnp.float32),
      ],
  )
  def edge_encoder(ea_hbm, w0_hbm, w1_hbm, w2_hbm, out_hbm,
                   w0_v, w1_v, w2_v, c_v, e_v, o_v):
    cid = lax.axis_index("c")
    sid = lax.axis_index("s")
    wid = sid * num_cores + cid

    pltpu.sync_copy(w0_hbm, w0_v)
    pltpu.sync_copy(w1_hbm, w1_v)
    pltpu.sync_copy(w2_hbm, w2_v)
    for i0 in range(D0):
      for i1 in range(D1):
        for i2 in range(D2):
          row = (i0 * D1 + i1) * D2 + i2
          for h in range(EMB // L):
            ds = pl.ds(h * L, L)
            c_v[row, ds] = w0_v[i0, ds] + w1_v[i1, ds] + w2_v[i2, ds]

    iota = lax.iota(jnp.int32, L)
    base0 = wid * per_w

    def chunk_body(k, _):
      base = base0 + k * B
      pltpu.sync_copy(ea_hbm.at[pl.ds(base, B), :], e_v)

      def group_body(g, _):
        rows = iota + g * L
        zero = jnp.zeros((L,), jnp.int32)
        e0 = plsc.load_gather(e_v, [rows, zero])
        e1 = plsc.load_gather(e_v, [rows, zero + 1])
        e2 = plsc.load_gather(e_v, [rows, zero + 2])
        c = (e0 * D1 + e1) * D2 + e2
        for d in range(EMB):
          dcol = jnp.full((L,), d, jnp.int32)
          v = plsc.load_gather(c_v, [c, dcol])
          plsc.store_scatter(o_v, [rows, dcol], v)
        return 0

      lax.fori_loop(0, B // L, group_body, 0)
      pltpu.sync_copy(o_v, out_hbm.at[pl.ds(base, B), :])
      return 0

    lax.fori_loop(0, n_chunks, chunk_body, 0)

  return edge_encoder


def kernel(edge_attr, W0, W1, W2):
  info = plsc.get_sparse_core_info()
  fn = _make_kernel(info.num_cores, info.num_subcores)
  return fn(edge_attr.astype(jnp.int32), W0, W1, W2)
